# baseline (speedup 1.0000x reference)
import functools

import jax
import jax.numpy as jnp
from jax import lax
from jax.experimental import pallas as pl
from jax.experimental.pallas import tpu as pltpu
from jax.experimental.pallas import tpu_sc as plsc

_H = 512
_W = 512
_NCMP = 20000
_NC = 2
_NS = 16
_NW = _NC * _NS
_L = 16
_PW = _NCMP // _NW
_NV = (_PW + _L - 1) // _L
_NCH = 5
_TGT_N = _NCMP * 6
_WN = 3760


def _whdr_partials(img_flat, tgt_flat):
    mesh = plsc.VectorSubcoreMesh(core_axis_name="c", subcore_axis_name="s")

    @functools.partial(
        pl.kernel,
        mesh=mesh,
        compiler_params=pltpu.CompilerParams(needs_layout_passes=False),
        out_type=jax.ShapeDtypeStruct((_NC, 2, _L), jnp.float32),
        scratch_types=[
            pltpu.VMEM((_WN,), jnp.float32),
            pltpu.VMEM((2 * _NCH, 128), jnp.int32),
            pltpu.VMEM((2 * _NCH, 128), jnp.float32),
            pltpu.VMEM((_L,), jnp.int32),
            pltpu.VMEM((_L,), jnp.float32),
            pltpu.VMEM((_L,), jnp.float32),
            pltpu.VMEM_SHARED((_L,), jnp.float32),
            pltpu.VMEM_SHARED((_L,), jnp.float32),
            pltpu.SemaphoreType.DMA,
        ],
    )
    def whdr_kernel(img_hbm, tgt_hbm, out_hbm, tv, idxv, pixv, iotav,
                    numv, denv, sh_num, sh_den, sem):
        c = lax.axis_index("c")
        s = lax.axis_index("s")
        wid = s * _NC + c

        zeros = jnp.zeros((_L,), jnp.float32)
        numv[...] = zeros

        @pl.when(s == 0)
        def _init_shared():
            pltpu.sync_copy(numv, sh_num)
            pltpu.sync_copy(numv, sh_den)

        lo6 = wid * (_PW * 6)
        base = jnp.minimum((lo6 // 8) * 8, _TGT_N - _WN)
        base = pl.multiple_of(base, 8)
        pltpu.sync_copy(tgt_hbm.at[pl.ds(base, _WN)], tv)
        d = lo6 - base

        iota = lax.iota(jnp.int32, _L)
        iota6 = iota * 6

        def field(v, f):
            idx = iota6 + (d + (96 * v + f))
            if v == _NV - 1:
                idx = jnp.minimum(idx, _WN - 1)
            return plsc.load_gather(tv, [idx])

        valid_tail = iota < (_PW - (_NV - 1) * _L)

        copies = [None] * (2 * _NCH)
        for k in range(_NCH):
            for u in range(8):
                v = 8 * k + u
                x1 = (field(v, 2) * _W).astype(jnp.int32)
                y1 = (field(v, 3) * _H).astype(jnp.int32)
                x2 = (field(v, 4) * _W).astype(jnp.int32)
                y2 = (field(v, 5) * _H).astype(jnp.int32)
                i1 = y1 * _W + x1
                i2 = y2 * _W + x2
                if v == _NV - 1:
                    i1 = jnp.where(valid_tail, i1, 0)
                    i2 = jnp.where(valid_tail, i2, 0)
                sl = pl.ds(u * _L, _L)
                idxv[2 * k, sl] = i1
                idxv[2 * k + 1, sl] = i2
            for j in (2 * k, 2 * k + 1):
                copies[j] = pltpu.async_copy(
                    img_hbm.at[idxv.at[j]], pixv.at[j], sem)

        hi = 1.0 + 0.12
        lo = 1.0 / (1.0 + 0.12)
        num = zeros
        den = zeros
        for k in range(_NCH):
            copies[2 * k].wait()
            copies[2 * k + 1].wait()
            for u in range(8):
                v = 8 * k + u
                sl = pl.ds(u * _L, _L)
                r1 = pixv[2 * k, sl]
                r2 = pixv[2 * k + 1, sl]
                ratio = r1 / (r2 + 1e-07)
                pred = jnp.where(ratio > hi, 2.0,
                                 jnp.where(ratio < lo, 1.0, 0.0))
                wt = field(v, 0)
                lab = field(v, 1)
                if v == _NV - 1:
                    wt = jnp.where(valid_tail, wt, 0.0)
                num = num + jnp.where(lab != pred, wt, zeros)
                den = den + wt
        numv[...] = num
        denv[...] = den
        iotav[...] = iota

        plsc.subcore_barrier()
        pltpu.sync_copy(numv, sh_num.at[iotav], add=True)
        pltpu.sync_copy(denv, sh_den.at[iotav], add=True)
        plsc.subcore_barrier()

        @pl.when(s == 0)
        def _finish():
            pltpu.sync_copy(sh_num, out_hbm.at[c, 0])
            pltpu.sync_copy(sh_den, out_hbm.at[c, 1])

    return whdr_kernel(img_flat, tgt_flat)


def kernel(input, target):
    img_flat = input.reshape(_H * _W)
    tgt_flat = target.reshape(_TGT_N)
    parts = _whdr_partials(img_flat, tgt_flat)
    num = jnp.sum(parts[:, 0, :])
    den = jnp.sum(parts[:, 1, :])
    return (num / den).reshape(1)


# speedup vs baseline: 1.4319x; 1.4319x over previous
import functools

import jax
import jax.numpy as jnp
from jax import lax
from jax.experimental import pallas as pl
from jax.experimental.pallas import tpu as pltpu
from jax.experimental.pallas import tpu_sc as plsc

_H = 512
_W = 512
_NCMP = 20000
_NC = 2
_NS = 16
_NW = _NC * _NS
_L = 16
_NPAD = 20480
_PER_W = _NPAD // _NW
_NV = _PER_W // _L
_NCH = 5


def _whdr_partials(img_flat, tgt):
    mesh = plsc.VectorSubcoreMesh(core_axis_name="c", subcore_axis_name="s")

    @functools.partial(
        pl.kernel,
        mesh=mesh,
        out_type=jax.ShapeDtypeStruct((_NC, 2, _L), jnp.float32),
        scratch_types=[
            pltpu.VMEM((6, _PER_W), jnp.float32),
            pltpu.VMEM((2 * _NCH, 128), jnp.int32),
            pltpu.VMEM((2 * _NCH, 128), jnp.float32),
            pltpu.VMEM((_L,), jnp.int32),
            pltpu.VMEM((_L,), jnp.float32),
            pltpu.VMEM((_L,), jnp.float32),
            pltpu.VMEM_SHARED((_L,), jnp.float32),
            pltpu.VMEM_SHARED((_L,), jnp.float32),
            pltpu.SemaphoreType.DMA,
        ],
    )
    def whdr_kernel(img_hbm, tgt_hbm, out_hbm, tv, idxv, pixv, iotav,
                    numv, denv, sh_num, sh_den, sem):
        c = lax.axis_index("c")
        s = lax.axis_index("s")
        wid = s * _NC + c
        base = wid * _PER_W

        zeros = jnp.zeros((_L,), jnp.float32)
        numv[...] = zeros

        @pl.when(s == 0)
        def _init_shared():
            pltpu.sync_copy(numv, sh_num)
            pltpu.sync_copy(numv, sh_den)

        for r in range(6):
            pltpu.sync_copy(tgt_hbm.at[r, pl.ds(base, _PER_W)], tv.at[r])

        copies = [None] * (2 * _NCH)
        for k in range(_NCH):
            for u in range(8):
                v = 8 * k + u
                sl = pl.ds(v * _L, _L)
                usl = pl.ds(u * _L, _L)
                x1 = (tv[2, sl] * _W).astype(jnp.int32)
                y1 = (tv[3, sl] * _H).astype(jnp.int32)
                x2 = (tv[4, sl] * _W).astype(jnp.int32)
                y2 = (tv[5, sl] * _H).astype(jnp.int32)
                idxv[2 * k, usl] = y1 * _W + x1
                idxv[2 * k + 1, usl] = y2 * _W + x2
            for j in (2 * k, 2 * k + 1):
                copies[j] = pltpu.async_copy(
                    img_hbm.at[idxv.at[j]], pixv.at[j], sem)

        hi = 1.0 + 0.12
        lo = 1.0 / (1.0 + 0.12)
        num = zeros
        den = zeros
        for k in range(_NCH):
            copies[2 * k].wait()
            copies[2 * k + 1].wait()
            for u in range(8):
                v = 8 * k + u
                sl = pl.ds(v * _L, _L)
                usl = pl.ds(u * _L, _L)
                r1 = pixv[2 * k, usl]
                r2 = pixv[2 * k + 1, usl]
                ratio = r1 / (r2 + 1e-07)
                pred = jnp.where(ratio > hi, 2.0,
                                 jnp.where(ratio < lo, 1.0, 0.0))
                wt = tv[0, sl]
                lab = tv[1, sl]
                num = num + jnp.where(lab != pred, wt, zeros)
                den = den + wt
        numv[...] = num
        denv[...] = den
        iotav[...] = lax.iota(jnp.int32, _L)

        plsc.subcore_barrier()
        pltpu.sync_copy(numv, sh_num.at[iotav], add=True)
        pltpu.sync_copy(denv, sh_den.at[iotav], add=True)
        plsc.subcore_barrier()

        @pl.when(s == 0)
        def _finish():
            pltpu.sync_copy(sh_num, out_hbm.at[c, 0])
            pltpu.sync_copy(sh_den, out_hbm.at[c, 1])

    return whdr_kernel(img_flat, tgt)


def kernel(input, target):
    img_flat = input.reshape(_H * _W)
    tpad = jnp.pad(target[0], ((0, _NPAD - _NCMP), (0, 0)))
    parts = _whdr_partials(img_flat, tpad.T)
    num = jnp.sum(parts[:, 0, :])
    den = jnp.sum(parts[:, 1, :])
    return (num / den).reshape(1)


# speedup vs baseline: 1.5339x; 1.0713x over previous
import functools

import jax
import jax.numpy as jnp
from jax import lax
from jax.experimental import pallas as pl
from jax.experimental.pallas import tpu as pltpu
from jax.experimental.pallas import tpu_sc as plsc

_H = 512
_W = 512
_NCMP = 20000
_NC = 2
_NS = 16
_NW = _NC * _NS
_L = 16
_NPAD = 20480
_PER_W = _NPAD // _NW
_NV = _PER_W // _L
_NCH = 5


def _whdr_partials(img_flat, tgt):
    mesh = plsc.VectorSubcoreMesh(core_axis_name="c", subcore_axis_name="s")

    @functools.partial(
        pl.kernel,
        mesh=mesh,
        out_type=jax.ShapeDtypeStruct((_NC, 2, _L), jnp.float32),
        scratch_types=[
            pltpu.VMEM((6, _PER_W), jnp.float32),
            pltpu.VMEM((2 * _NCH, 128), jnp.int32),
            pltpu.VMEM((2 * _NCH, 128), jnp.float32),
            pltpu.VMEM((_L,), jnp.int32),
            pltpu.VMEM((_L,), jnp.float32),
            pltpu.VMEM((_L,), jnp.float32),
            pltpu.VMEM_SHARED((_L,), jnp.float32),
            pltpu.VMEM_SHARED((_L,), jnp.float32),
            pltpu.SemaphoreType.DMA,
            pltpu.SemaphoreType.DMA,
        ],
    )
    def whdr_kernel(img_hbm, tgt_hbm, out_hbm, tv, idxv, pixv, iotav,
                    numv, denv, sh_num, sh_den, sem, sem2):
        c = lax.axis_index("c")
        s = lax.axis_index("s")
        wid = s * _NC + c
        base = wid * _PER_W

        tcp = pltpu.async_copy(tgt_hbm.at[:, pl.ds(base, _PER_W)], tv, sem2)

        zeros = jnp.zeros((_L,), jnp.float32)
        numv[...] = zeros
        iotav[...] = lax.iota(jnp.int32, _L)

        @pl.when(s == 0)
        def _init_shared():
            pltpu.sync_copy(numv, sh_num)
            pltpu.sync_copy(numv, sh_den)

        tcp.wait()

        copies = [None] * (2 * _NCH)
        for k in range(_NCH):
            for u in range(8):
                v = 8 * k + u
                sl = pl.ds(v * _L, _L)
                usl = pl.ds(u * _L, _L)
                x1 = (tv[2, sl] * _W).astype(jnp.int32)
                y1 = (tv[3, sl] * _H).astype(jnp.int32)
                x2 = (tv[4, sl] * _W).astype(jnp.int32)
                y2 = (tv[5, sl] * _H).astype(jnp.int32)
                idxv[2 * k, usl] = y1 * _W + x1
                idxv[2 * k + 1, usl] = y2 * _W + x2
            for j in (2 * k, 2 * k + 1):
                copies[j] = pltpu.async_copy(
                    img_hbm.at[idxv.at[j]], pixv.at[j], sem)

        hi = 1.0 + 0.12
        lo = 1.0 / (1.0 + 0.12)
        num = zeros
        den = zeros
        for k in range(_NCH):
            copies[2 * k].wait()
            copies[2 * k + 1].wait()
            for u in range(8):
                v = 8 * k + u
                sl = pl.ds(v * _L, _L)
                usl = pl.ds(u * _L, _L)
                r1 = pixv[2 * k, usl]
                r2 = pixv[2 * k + 1, usl]
                ratio = r1 / (r2 + 1e-07)
                pred = jnp.where(ratio > hi, 2.0,
                                 jnp.where(ratio < lo, 1.0, 0.0))
                wt = tv[0, sl]
                lab = tv[1, sl]
                num = num + jnp.where(lab != pred, wt, zeros)
                den = den + wt
        numv[...] = num
        denv[...] = den

        plsc.subcore_barrier()
        acp1 = pltpu.async_copy(numv, sh_num.at[iotav], sem2, add=True)
        acp2 = pltpu.async_copy(denv, sh_den.at[iotav], sem2, add=True)
        acp1.wait()
        acp2.wait()
        plsc.subcore_barrier()

        @pl.when(s == 0)
        def _finish():
            pltpu.sync_copy(sh_num, out_hbm.at[c, 0])
            pltpu.sync_copy(sh_den, out_hbm.at[c, 1])

    return whdr_kernel(img_flat, tgt)


def kernel(input, target):
    img_flat = input.reshape(_H * _W)
    tpad = jnp.pad(target[0], ((0, _NPAD - _NCMP), (0, 0)))
    parts = _whdr_partials(img_flat, tpad.T)
    num = jnp.sum(parts[:, 0, :])
    den = jnp.sum(parts[:, 1, :])
    return (num / den).reshape(1)


# speedup vs baseline: 1.7363x; 1.1319x over previous
import functools

import jax
import jax.numpy as jnp
from jax import lax
from jax.experimental import pallas as pl
from jax.experimental.pallas import tpu as pltpu
from jax.experimental.pallas import tpu_sc as plsc

_H = 512
_W = 512
_NCMP = 20000
_NC = 2
_NS = 16
_NW = _NC * _NS
_L = 16
_NPAD = 20480
_PER_W = _NPAD // _NW
_NV = _PER_W // _L
_NCH = 5


def _whdr_partials(img_flat, tgt):
    mesh = plsc.VectorSubcoreMesh(core_axis_name="c", subcore_axis_name="s")

    @functools.partial(
        pl.kernel,
        mesh=mesh,
        out_type=jax.ShapeDtypeStruct((_NC, 2, _L), jnp.float32),
        scratch_types=[
            pltpu.VMEM((6, _PER_W), jnp.float32),
            pltpu.VMEM((2 * _NCH, 128), jnp.int32),
            pltpu.VMEM((2 * _NCH, 128), jnp.float32),
            pltpu.VMEM((_L,), jnp.int32),
            pltpu.VMEM((_L,), jnp.float32),
            pltpu.VMEM((_L,), jnp.float32),
            pltpu.VMEM_SHARED((_L,), jnp.float32),
            pltpu.VMEM_SHARED((_L,), jnp.float32),
            pltpu.VMEM_SHARED((_H * _W,), jnp.float32),
            pltpu.SemaphoreType.DMA,
            pltpu.SemaphoreType.DMA,
        ],
    )
    def whdr_kernel(img_hbm, tgt_hbm, out_hbm, tv, idxv, pixv, iotav,
                    numv, denv, sh_num, sh_den, sh_img, sem, sem2):
        c = lax.axis_index("c")
        s = lax.axis_index("s")
        wid = s * _NC + c
        base = wid * _PER_W

        ichunk = (_H * _W) // _NS
        isl = pl.ds(s * ichunk, ichunk)
        icp = pltpu.async_copy(img_hbm.at[isl], sh_img.at[isl], sem2)
        tcp = pltpu.async_copy(tgt_hbm.at[:, pl.ds(base, _PER_W)], tv, sem2)

        zeros = jnp.zeros((_L,), jnp.float32)
        numv[...] = zeros
        iotav[...] = lax.iota(jnp.int32, _L)

        @pl.when(s == 0)
        def _init_shared():
            pltpu.sync_copy(numv, sh_num)
            pltpu.sync_copy(numv, sh_den)

        tcp.wait()

        copies = [None] * (2 * _NCH)
        fired = False
        for k in range(_NCH):
            for u in range(8):
                v = 8 * k + u
                sl = pl.ds(v * _L, _L)
                usl = pl.ds(u * _L, _L)
                x1 = (tv[2, sl] * _W).astype(jnp.int32)
                y1 = (tv[3, sl] * _H).astype(jnp.int32)
                x2 = (tv[4, sl] * _W).astype(jnp.int32)
                y2 = (tv[5, sl] * _H).astype(jnp.int32)
                idxv[2 * k, usl] = y1 * _W + x1
                idxv[2 * k + 1, usl] = y2 * _W + x2
            if not fired:
                icp.wait()
                plsc.subcore_barrier()
                fired = True
            for j in (2 * k, 2 * k + 1):
                copies[j] = pltpu.async_copy(
                    sh_img.at[idxv.at[j]], pixv.at[j], sem)

        hi = 1.0 + 0.12
        lo = 1.0 / (1.0 + 0.12)
        num = zeros
        den = zeros
        for k in range(_NCH):
            copies[2 * k].wait()
            copies[2 * k + 1].wait()
            for u in range(8):
                v = 8 * k + u
                sl = pl.ds(v * _L, _L)
                usl = pl.ds(u * _L, _L)
                r1 = pixv[2 * k, usl]
                r2 = pixv[2 * k + 1, usl]
                ratio = r1 / (r2 + 1e-07)
                pred = jnp.where(ratio > hi, 2.0,
                                 jnp.where(ratio < lo, 1.0, 0.0))
                wt = tv[0, sl]
                lab = tv[1, sl]
                num = num + jnp.where(lab != pred, wt, zeros)
                den = den + wt
        numv[...] = num
        denv[...] = den

        plsc.subcore_barrier()
        acp1 = pltpu.async_copy(numv, sh_num.at[iotav], sem2, add=True)
        acp2 = pltpu.async_copy(denv, sh_den.at[iotav], sem2, add=True)
        acp1.wait()
        acp2.wait()
        plsc.subcore_barrier()

        @pl.when(s == 0)
        def _finish():
            pltpu.sync_copy(sh_num, out_hbm.at[c, 0])
            pltpu.sync_copy(sh_den, out_hbm.at[c, 1])

    return whdr_kernel(img_flat, tgt)


def kernel(input, target):
    img_flat = input.reshape(_H * _W)
    tpad = jnp.pad(target[0], ((0, _NPAD - _NCMP), (0, 0)))
    parts = _whdr_partials(img_flat, tpad.T)
    num = jnp.sum(parts[:, 0, :])
    den = jnp.sum(parts[:, 1, :])
    return (num / den).reshape(1)


# speedup vs baseline: 1.7373x; 1.0006x over previous
import functools

import jax
import jax.numpy as jnp
from jax import lax
from jax.experimental import pallas as pl
from jax.experimental.pallas import tpu as pltpu
from jax.experimental.pallas import tpu_sc as plsc

_H = 512
_W = 512
_NCMP = 20000
_NC = 2
_NS = 16
_NW = _NC * _NS
_L = 16
_NPAD = 20480
_PER_W = _NPAD // _NW
_NV = _PER_W // _L
_NCH = 5


def _whdr_partials(img_flat, tgt):
    mesh = plsc.VectorSubcoreMesh(core_axis_name="c", subcore_axis_name="s")

    @functools.partial(
        pl.kernel,
        mesh=mesh,
        compiler_params=pltpu.CompilerParams(skip_device_barrier=True),
        out_type=jax.ShapeDtypeStruct((_NC, 2, _L), jnp.float32),
        scratch_types=[
            pltpu.VMEM((6, _PER_W), jnp.float32),
            pltpu.VMEM((2 * _NCH, 128), jnp.int32),
            pltpu.VMEM((2 * _NCH, 128), jnp.float32),
            pltpu.VMEM((_L,), jnp.int32),
            pltpu.VMEM((_L,), jnp.float32),
            pltpu.VMEM((_L,), jnp.float32),
            pltpu.VMEM_SHARED((_L,), jnp.float32),
            pltpu.VMEM_SHARED((_L,), jnp.float32),
            pltpu.VMEM_SHARED((_H * _W,), jnp.float32),
            pltpu.SemaphoreType.DMA,
            pltpu.SemaphoreType.DMA,
        ],
    )
    def whdr_kernel(img_hbm, tgt_hbm, out_hbm, tv, idxv, pixv, iotav,
                    numv, denv, sh_num, sh_den, sh_img, sem, sem2):
        c = lax.axis_index("c")
        s = lax.axis_index("s")
        wid = s * _NC + c
        base = wid * _PER_W

        ichunk = (_H * _W) // _NS
        isl = pl.ds(s * ichunk, ichunk)
        icp = pltpu.async_copy(img_hbm.at[isl], sh_img.at[isl], sem2)
        tcp = pltpu.async_copy(tgt_hbm.at[:, pl.ds(base, _PER_W)], tv, sem2)

        zeros = jnp.zeros((_L,), jnp.float32)
        numv[...] = zeros
        iotav[...] = lax.iota(jnp.int32, _L)

        @pl.when(s == 0)
        def _init_shared():
            pltpu.sync_copy(numv, sh_num)
            pltpu.sync_copy(numv, sh_den)

        tcp.wait()

        copies = [None] * (2 * _NCH)
        fired = False
        for k in range(_NCH):
            for u in range(8):
                v = 8 * k + u
                sl = pl.ds(v * _L, _L)
                usl = pl.ds(u * _L, _L)
                x1 = (tv[2, sl] * _W).astype(jnp.int32)
                y1 = (tv[3, sl] * _H).astype(jnp.int32)
                x2 = (tv[4, sl] * _W).astype(jnp.int32)
                y2 = (tv[5, sl] * _H).astype(jnp.int32)
                idxv[2 * k, usl] = y1 * _W + x1
                idxv[2 * k + 1, usl] = y2 * _W + x2
            if not fired:
                icp.wait()
                plsc.subcore_barrier()
                fired = True
            for j in (2 * k, 2 * k + 1):
                copies[j] = pltpu.async_copy(
                    sh_img.at[idxv.at[j]], pixv.at[j], sem)

        hi = 1.0 + 0.12
        lo = 1.0 / (1.0 + 0.12)
        num = zeros
        den = zeros
        for k in range(_NCH):
            copies[2 * k].wait()
            copies[2 * k + 1].wait()
            for u in range(8):
                v = 8 * k + u
                sl = pl.ds(v * _L, _L)
                usl = pl.ds(u * _L, _L)
                r1 = pixv[2 * k, usl]
                r2 = pixv[2 * k + 1, usl]
                ratio = r1 / (r2 + 1e-07)
                pred = jnp.where(ratio > hi, 2.0,
                                 jnp.where(ratio < lo, 1.0, 0.0))
                wt = tv[0, sl]
                lab = tv[1, sl]
                num = num + jnp.where(lab != pred, wt, zeros)
                den = den + wt
        numv[...] = num
        denv[...] = den

        plsc.subcore_barrier()
        acp1 = pltpu.async_copy(numv, sh_num.at[iotav], sem2, add=True)
        acp2 = pltpu.async_copy(denv, sh_den.at[iotav], sem2, add=True)
        acp1.wait()
        acp2.wait()
        plsc.subcore_barrier()

        @pl.when(s == 0)
        def _finish():
            pltpu.sync_copy(sh_num, out_hbm.at[c, 0])
            pltpu.sync_copy(sh_den, out_hbm.at[c, 1])

    return whdr_kernel(img_flat, tgt)


def kernel(input, target):
    img_flat = input.reshape(_H * _W)
    tpad = jnp.pad(target[0], ((0, _NPAD - _NCMP), (0, 0)))
    parts = _whdr_partials(img_flat, tpad.T)
    num = jnp.sum(parts[:, 0, :])
    den = jnp.sum(parts[:, 1, :])
    return (num / den).reshape(1)


# speedup vs baseline: 1.7843x; 1.0270x over previous
import functools

import jax
import jax.numpy as jnp
from jax import lax
from jax.experimental import pallas as pl
from jax.experimental.pallas import tpu as pltpu
from jax.experimental.pallas import tpu_sc as plsc

_H = 512
_W = 512
_NCMP = 20000
_NC = 2
_NS = 16
_NW = _NC * _NS
_L = 16
_NPAD = 20480
_PER_W = _NPAD // _NW
_NV = _PER_W // _L
_NCH = 5


def _whdr_partials(img_flat, tgt):
    mesh = plsc.VectorSubcoreMesh(core_axis_name="c", subcore_axis_name="s")

    @functools.partial(
        pl.kernel,
        mesh=mesh,
        out_type=jax.ShapeDtypeStruct((_NC, _NS, 2, _L), jnp.float32),
        scratch_types=[
            pltpu.VMEM((6, _PER_W), jnp.float32),
            pltpu.VMEM((2 * _NCH, 128), jnp.int32),
            pltpu.VMEM((2 * _NCH, 128), jnp.float32),
            pltpu.VMEM((_L,), jnp.float32),
            pltpu.VMEM((_L,), jnp.float32),
            pltpu.VMEM_SHARED((_H * _W,), jnp.float32),
            pltpu.SemaphoreType.DMA,
            pltpu.SemaphoreType.DMA,
        ],
    )
    def whdr_kernel(img_hbm, tgt_hbm, out_hbm, tv, idxv, pixv,
                    numv, denv, sh_img, sem, sem2):
        c = lax.axis_index("c")
        s = lax.axis_index("s")
        wid = s * _NC + c
        base = wid * _PER_W

        ichunk = (_H * _W) // _NS
        isl = pl.ds(s * ichunk, ichunk)
        icp = pltpu.async_copy(img_hbm.at[isl], sh_img.at[isl], sem2)
        tcp = pltpu.async_copy(tgt_hbm.at[:, pl.ds(base, _PER_W)], tv, sem2)

        zeros = jnp.zeros((_L,), jnp.float32)
        tcp.wait()

        copies = [None] * (2 * _NCH)
        fired = False
        for k in range(_NCH):
            for u in range(8):
                v = 8 * k + u
                sl = pl.ds(v * _L, _L)
                usl = pl.ds(u * _L, _L)
                x1 = (tv[2, sl] * _W).astype(jnp.int32)
                y1 = (tv[3, sl] * _H).astype(jnp.int32)
                x2 = (tv[4, sl] * _W).astype(jnp.int32)
                y2 = (tv[5, sl] * _H).astype(jnp.int32)
                idxv[2 * k, usl] = y1 * _W + x1
                idxv[2 * k + 1, usl] = y2 * _W + x2
            if not fired:
                icp.wait()
                plsc.subcore_barrier()
                fired = True
            for j in (2 * k, 2 * k + 1):
                copies[j] = pltpu.async_copy(
                    sh_img.at[idxv.at[j]], pixv.at[j], sem)

        hi = 1.0 + 0.12
        lo = 1.0 / (1.0 + 0.12)
        num = zeros
        den = zeros
        for k in range(_NCH):
            copies[2 * k].wait()
            copies[2 * k + 1].wait()
            for u in range(8):
                v = 8 * k + u
                sl = pl.ds(v * _L, _L)
                usl = pl.ds(u * _L, _L)
                r1 = pixv[2 * k, usl]
                r2 = pixv[2 * k + 1, usl]
                ratio = r1 / (r2 + 1e-07)
                pred = jnp.where(ratio > hi, 2.0,
                                 jnp.where(ratio < lo, 1.0, 0.0))
                wt = tv[0, sl]
                lab = tv[1, sl]
                num = num + jnp.where(lab != pred, wt, zeros)
                den = den + wt
        numv[...] = num
        denv[...] = den

        acp1 = pltpu.async_copy(numv, out_hbm.at[c, s, 0], sem2)
        acp2 = pltpu.async_copy(denv, out_hbm.at[c, s, 1], sem2)
        acp1.wait()
        acp2.wait()

    return whdr_kernel(img_flat, tgt)


def kernel(input, target):
    img_flat = input.reshape(_H * _W)
    tpad = jnp.pad(target[0], ((0, _NPAD - _NCMP), (0, 0)))
    parts = _whdr_partials(img_flat, tpad.T)
    num = jnp.sum(parts[:, :, 0, :])
    den = jnp.sum(parts[:, :, 1, :])
    return (num / den).reshape(1)


# speedup vs baseline: 1.7928x; 1.0048x over previous
import functools

import jax
import jax.numpy as jnp
from jax import lax
from jax.experimental import pallas as pl
from jax.experimental.pallas import tpu as pltpu
from jax.experimental.pallas import tpu_sc as plsc

_H = 512
_W = 512
_NCMP = 20000
_NC = 2
_NS = 16
_NW = _NC * _NS
_L = 16
_NPAD = 20480
_PER_W = _NPAD // _NW
_NV = _PER_W // _L
_NCH = 5


def _whdr_partials(img_flat, tgt):
    mesh = plsc.VectorSubcoreMesh(core_axis_name="c", subcore_axis_name="s")

    @functools.partial(
        pl.kernel,
        mesh=mesh,
        out_type=jax.ShapeDtypeStruct((_NC, _NS, 2, _L), jnp.float32),
        scratch_types=[
            pltpu.VMEM((6, _PER_W), jnp.float32),
            pltpu.VMEM((2 * _NCH, 128), jnp.int32),
            pltpu.VMEM((2 * _NCH, 128), jnp.float32),
            pltpu.VMEM((_L,), jnp.float32),
            pltpu.VMEM((_L,), jnp.float32),
            pltpu.VMEM_SHARED((_H * _W,), jnp.float32),
            pltpu.SemaphoreType.DMA,
            pltpu.SemaphoreType.DMA,
        ],
    )
    def whdr_kernel(img_hbm, tgt_hbm, out_hbm, tv, idxv, pixv,
                    numv, denv, sh_img, sem, sem2):
        c = lax.axis_index("c")
        s = lax.axis_index("s")
        wid = s * _NC + c
        base = wid * _PER_W

        ichunk = (_H * _W) // _NS
        isl = pl.ds(s * ichunk, ichunk)
        icp = pltpu.async_copy(img_hbm.at[isl], sh_img.at[isl], sem2)
        tcp = pltpu.async_copy(tgt_hbm.at[:, pl.ds(base, _PER_W)], tv, sem2)

        zeros = jnp.zeros((_L,), jnp.float32)
        tcp.wait()

        for k in range(_NCH):
            for u in range(8):
                v = 8 * k + u
                sl = pl.ds(v * _L, _L)
                usl = pl.ds(u * _L, _L)
                x1 = (tv[2, sl] * _W).astype(jnp.int32)
                y1 = (tv[3, sl] * _H).astype(jnp.int32)
                x2 = (tv[4, sl] * _W).astype(jnp.int32)
                y2 = (tv[5, sl] * _H).astype(jnp.int32)
                idxv[2 * k, usl] = y1 * _W + x1
                idxv[2 * k + 1, usl] = y2 * _W + x2

        icp.wait()
        plsc.subcore_barrier()
        copies = [pltpu.async_copy(sh_img.at[idxv.at[j]], pixv.at[j], sem)
                  for j in range(2 * _NCH)]

        hi = 1.0 + 0.12
        lo = 1.0 / (1.0 + 0.12)
        num = zeros
        den = zeros
        for k in range(_NCH):
            copies[2 * k].wait()
            copies[2 * k + 1].wait()
            for u in range(8):
                v = 8 * k + u
                sl = pl.ds(v * _L, _L)
                usl = pl.ds(u * _L, _L)
                r1 = pixv[2 * k, usl]
                r2 = pixv[2 * k + 1, usl]
                r2e = r2 + 1e-07
                pred = jnp.where(r1 > hi * r2e, 2.0,
                                 jnp.where(r1 < lo * r2e, 1.0, 0.0))
                wt = tv[0, sl]
                lab = tv[1, sl]
                num = num + jnp.where(lab != pred, wt, zeros)
                den = den + wt
        numv[...] = num
        denv[...] = den

        acp1 = pltpu.async_copy(numv, out_hbm.at[c, s, 0], sem2)
        acp2 = pltpu.async_copy(denv, out_hbm.at[c, s, 1], sem2)
        acp1.wait()
        acp2.wait()

    return whdr_kernel(img_flat, tgt)


def kernel(input, target):
    img_flat = input.reshape(_H * _W)
    tpad = jnp.pad(target[0], ((0, _NPAD - _NCMP), (0, 0)))
    parts = _whdr_partials(img_flat, tpad.T)
    num = jnp.sum(parts[:, :, 0, :])
    den = jnp.sum(parts[:, :, 1, :])
    return (num / den).reshape(1)


# speedup vs baseline: 1.9706x; 1.0992x over previous
import functools

import jax
import jax.numpy as jnp
from jax import lax
from jax.experimental import pallas as pl
from jax.experimental.pallas import tpu as pltpu
from jax.experimental.pallas import tpu_sc as plsc

_H = 512
_W = 512
_NCMP = 20000
_NS = 16
_L = 16
_NPAD = 20480
_PER_W = _NPAD // _NS
_NV = _PER_W // _L
_NCH = _PER_W // 128


def _whdr_scalar(img_flat, tgt):
    mesh = plsc.VectorSubcoreMesh(core_axis_name="c", subcore_axis_name="s",
                                  num_cores=1)

    @functools.partial(
        pl.kernel,
        mesh=mesh,
        out_type=jax.ShapeDtypeStruct((1,), jnp.float32),
        scratch_types=[
            pltpu.VMEM((6, _PER_W), jnp.float32),
            pltpu.VMEM((2 * _NCH, 128), jnp.int32),
            pltpu.VMEM((2 * _NCH, 128), jnp.float32),
            pltpu.VMEM((_L,), jnp.int32),
            pltpu.VMEM((_L,), jnp.float32),
            pltpu.VMEM((_L,), jnp.float32),
            pltpu.VMEM((_L,), jnp.float32),
            pltpu.VMEM_SHARED((_L,), jnp.float32),
            pltpu.VMEM_SHARED((_L,), jnp.float32),
            pltpu.VMEM_SHARED((_H * _W,), jnp.float32),
            pltpu.SemaphoreType.DMA,
            pltpu.SemaphoreType.DMA,
        ],
    )
    def whdr_kernel(img_hbm, tgt_hbm, out_hbm, tv, idxv, pixv, zidx,
                    numv, denv, quov, sh_n, sh_d, sh_img, sem, sem2):
        s = lax.axis_index("s")
        base = s * _PER_W

        ichunk = (_H * _W) // _NS
        isl = pl.ds(s * ichunk, ichunk)
        icp = pltpu.async_copy(img_hbm.at[isl], sh_img.at[isl], sem2)
        tcp = pltpu.async_copy(tgt_hbm.at[:, pl.ds(base, _PER_W)], tv, sem2)

        zeros = jnp.zeros((_L,), jnp.float32)
        zidx[...] = jnp.zeros((_L,), jnp.int32)
        numv[...] = zeros

        @pl.when(s == 0)
        def _init_shared():
            pltpu.sync_copy(numv, sh_n)
            pltpu.sync_copy(numv, sh_d)

        tcp.wait()

        for k in range(_NCH):
            for u in range(8):
                v = 8 * k + u
                sl = pl.ds(v * _L, _L)
                usl = pl.ds(u * _L, _L)
                x1 = (tv[2, sl] * _W).astype(jnp.int32)
                y1 = (tv[3, sl] * _H).astype(jnp.int32)
                x2 = (tv[4, sl] * _W).astype(jnp.int32)
                y2 = (tv[5, sl] * _H).astype(jnp.int32)
                idxv[2 * k, usl] = y1 * _W + x1
                idxv[2 * k + 1, usl] = y2 * _W + x2

        icp.wait()
        plsc.subcore_barrier()
        copies = [pltpu.async_copy(sh_img.at[idxv.at[j]], pixv.at[j], sem)
                  for j in range(2 * _NCH)]

        hi = 1.0 + 0.12
        lo = 1.0 / (1.0 + 0.12)
        num = zeros
        den = zeros
        for k in range(_NCH):
            copies[2 * k].wait()
            copies[2 * k + 1].wait()
            for u in range(8):
                v = 8 * k + u
                sl = pl.ds(v * _L, _L)
                usl = pl.ds(u * _L, _L)
                r1 = pixv[2 * k, usl]
                r2 = pixv[2 * k + 1, usl]
                r2e = r2 + 1e-07
                pred = jnp.where(r1 > hi * r2e, 2.0,
                                 jnp.where(r1 < lo * r2e, 1.0, 0.0))
                wt = tv[0, sl]
                lab = tv[1, sl]
                num = num + jnp.where(lab != pred, wt, zeros)
                den = den + wt
        numv[...] = num
        denv[...] = den

        acp1 = pltpu.async_copy(numv, sh_n.at[zidx], sem2, add=True)
        acp2 = pltpu.async_copy(denv, sh_d.at[zidx], sem2, add=True)
        acp1.wait()
        acp2.wait()
        plsc.subcore_barrier()

        @pl.when(s == 0)
        def _finish():
            pltpu.sync_copy(sh_n, numv)
            pltpu.sync_copy(sh_d, denv)
            lane = lax.iota(jnp.int32, _L)
            d_safe = jnp.where(lane == 0, denv[...], 1.0)
            quov[...] = numv[...] / d_safe
            pltpu.sync_copy(quov.at[pl.ds(0, 1)], out_hbm)

    return whdr_kernel(img_flat, tgt)


def kernel(input, target):
    img_flat = input.reshape(_H * _W)
    tpad = jnp.pad(target[0], ((0, _NPAD - _NCMP), (0, 0)))
    return _whdr_scalar(img_flat, tpad.T)
